# baseline (speedup 1.0000x reference)
import functools

import jax
import jax.numpy as jnp
from jax.experimental import pallas as pl
from jax.experimental.pallas import tpu as pltpu

_BR = 256
_NS = 4

_NT = (((1,), (1,)), ((), ()))


def _airtnn_body(u_ref, l_ref, xt_ref, wt_ref, wh_ref, out_ref,
                 stage_ref, cache_ref, ys_ref, ya_ref, yb_ref, sem_ref,
                 *, nblk, taps):
    mrefs = (u_ref, l_ref)

    def copy(m, i):
        return pltpu.make_async_copy(
            mrefs[m].at[pl.ds(i * _BR, _BR), :],
            stage_ref.at[(m * nblk + i) % _NS],
            sem_ref.at[m, i],
        )

    def land(m, i):
        copy(m, i).wait()
        s = m * nblk + i
        cache_ref[i] = stage_ref[s % _NS].astype(jnp.bfloat16)
        if s + _NS < 2 * nblk:
            copy(*divmod(s + _NS, nblk)).start()

    def tap(m, t, rdb, i, write_to=None):
        cols = pl.ds(i * _BR, _BR)
        yf = jax.lax.dot_general(rdb[:], cache_ref[i], _NT,
                                 preferred_element_type=jnp.float32)
        if write_to is not None:
            write_to[:, cols] = yf.astype(jnp.bfloat16)
        contrib = jax.lax.dot(wt_ref[m, t], yf,
                              preferred_element_type=jnp.float32)
        if m == 0 and t == 0:
            out_ref[:, cols] = contrib + jax.lax.dot(
                wh_ref[:], xt_ref[:, cols],
                preferred_element_type=jnp.float32)
        else:
            out_ref[:, cols] = out_ref[:, cols] + contrib

    for s in range(_NS):
        copy(*divmod(s, nblk)).start()

    ys_ref[:] = xt_ref[:].astype(jnp.bfloat16)

    for i in range(nblk):
        land(0, i)
        tap(0, 0, ys_ref, i, write_to=ya_ref)

    rdb, wrb = ya_ref, yb_ref
    for t in range(1, taps - 1):
        for i in range(nblk):
            tap(0, t, rdb, i, write_to=wrb)
        rdb, wrb = wrb, rdb

    for i in range(nblk):
        tap(0, taps - 1, rdb, i)
        land(1, i)
        tap(1, 0, ys_ref, i, write_to=ya_ref)

    rdb, wrb = ya_ref, yb_ref
    for t in range(1, taps):
        for i in range(nblk):
            tap(1, t, rdb, i, write_to=wrb if t < taps - 1 else None)
        rdb, wrb = wrb, rdb


def kernel(x, lower_lp, upper_lp, W_up, W_low, W_h):
    B, N, C_in = x.shape
    T, C_out, _ = W_up.shape
    BC = B * C_in
    BCO = B * C_out
    nblk = N // _BR

    xt = jnp.transpose(x, (0, 2, 1)).reshape(BC, N)

    eye = jnp.eye(B, dtype=jnp.float32)
    wt = jnp.stack([
        jnp.stack([jnp.kron(eye, W_up[t]) for t in range(T)]),
        jnp.stack([jnp.kron(eye, W_low[t]) for t in range(T)]),
    ])
    wh = jnp.kron(eye, W_h)

    out = pl.pallas_call(
        functools.partial(_airtnn_body, nblk=nblk, taps=T),
        in_specs=[
            pl.BlockSpec(memory_space=pl.ANY),
            pl.BlockSpec(memory_space=pl.ANY),
            pl.BlockSpec((BC, N), lambda: (0, 0)),
            pl.BlockSpec((2, T, BCO, BC), lambda: (0, 0, 0, 0)),
            pl.BlockSpec((BCO, BC), lambda: (0, 0)),
        ],
        out_specs=pl.BlockSpec((BCO, N), lambda: (0, 0)),
        out_shape=jax.ShapeDtypeStruct((BCO, N), jnp.float32),
        scratch_shapes=[
            pltpu.VMEM((_NS, _BR, N), jnp.float32),
            pltpu.VMEM((nblk, _BR, N), jnp.bfloat16),
            pltpu.VMEM((BC, N), jnp.bfloat16),
            pltpu.VMEM((BC, N), jnp.bfloat16),
            pltpu.VMEM((BC, N), jnp.bfloat16),
            pltpu.SemaphoreType.DMA((2, nblk)),
        ],
        compiler_params=pltpu.CompilerParams(
            vmem_limit_bytes=62 * 1024 * 1024,
        ),
    )(upper_lp, lower_lp, xt, wt, wh)

    return jnp.transpose(out.reshape(B, C_out, N), (0, 2, 1))


# speedup vs baseline: 1.3737x; 1.3737x over previous
import functools

import jax
import jax.numpy as jnp
from jax.experimental import pallas as pl
from jax.experimental.pallas import tpu as pltpu

_BR = 512
_NS = 2

_NT = (((1,), (1,)), ((), ()))


def _airtnn_body(u_ref, l_ref, xt_ref, wt_ref, wh_ref, out_ref,
                 stage_ref, cache_ref, ys_ref, ya_ref, yb_ref, sem_ref,
                 *, nblk, taps):
    mrefs = (u_ref, l_ref)

    def copy(m, i):
        return pltpu.make_async_copy(
            mrefs[m].at[pl.ds(i * _BR, _BR), :],
            stage_ref.at[(m * nblk + i) % _NS],
            sem_ref.at[m, i],
        )

    def land(m, i):
        copy(m, i).wait()
        s = m * nblk + i
        cache_ref[i] = stage_ref[s % _NS].astype(jnp.bfloat16)
        if s + _NS < 2 * nblk:
            copy(*divmod(s + _NS, nblk)).start()

    def tap(m, t, rdb, i, write_to=None):
        cols = pl.ds(i * _BR, _BR)
        yf = jax.lax.dot_general(rdb[:], cache_ref[i], _NT,
                                 preferred_element_type=jnp.float32)
        if write_to is not None:
            write_to[:, cols] = yf.astype(jnp.bfloat16)
        contrib = jax.lax.dot(wt_ref[m, t], yf,
                              preferred_element_type=jnp.float32)
        if m == 0 and t == 0:
            out_ref[:, cols] = contrib + jax.lax.dot(
                wh_ref[:], xt_ref[:, cols],
                preferred_element_type=jnp.float32)
        else:
            out_ref[:, cols] = out_ref[:, cols] + contrib

    for s in range(_NS):
        copy(*divmod(s, nblk)).start()

    ys_ref[:] = xt_ref[:].astype(jnp.bfloat16)

    for i in range(nblk):
        land(0, i)
        tap(0, 0, ys_ref, i, write_to=ya_ref)

    rdb, wrb = ya_ref, yb_ref
    for t in range(1, taps - 1):
        for i in range(nblk):
            tap(0, t, rdb, i, write_to=wrb)
        rdb, wrb = wrb, rdb

    for i in range(nblk):
        tap(0, taps - 1, rdb, i)
        land(1, i)
        tap(1, 0, ys_ref, i, write_to=ya_ref)

    rdb, wrb = ya_ref, yb_ref
    for t in range(1, taps):
        for i in range(nblk):
            tap(1, t, rdb, i, write_to=wrb if t < taps - 1 else None)
        rdb, wrb = wrb, rdb


def kernel(x, lower_lp, upper_lp, W_up, W_low, W_h):
    B, N, C_in = x.shape
    T, C_out, _ = W_up.shape
    BC = B * C_in
    BCO = B * C_out
    nblk = N // _BR

    xt = jnp.transpose(x, (0, 2, 1)).reshape(BC, N)

    eye = jnp.eye(B, dtype=jnp.float32)
    wt = jnp.stack([
        jnp.stack([jnp.kron(eye, W_up[t]) for t in range(T)]),
        jnp.stack([jnp.kron(eye, W_low[t]) for t in range(T)]),
    ])
    wh = jnp.kron(eye, W_h)

    out = pl.pallas_call(
        functools.partial(_airtnn_body, nblk=nblk, taps=T),
        in_specs=[
            pl.BlockSpec(memory_space=pl.ANY),
            pl.BlockSpec(memory_space=pl.ANY),
            pl.BlockSpec((BC, N), lambda: (0, 0)),
            pl.BlockSpec((2, T, BCO, BC), lambda: (0, 0, 0, 0)),
            pl.BlockSpec((BCO, BC), lambda: (0, 0)),
        ],
        out_specs=pl.BlockSpec((BCO, N), lambda: (0, 0)),
        out_shape=jax.ShapeDtypeStruct((BCO, N), jnp.float32),
        scratch_shapes=[
            pltpu.VMEM((_NS, _BR, N), jnp.float32),
            pltpu.VMEM((nblk, _BR, N), jnp.bfloat16),
            pltpu.VMEM((BC, N), jnp.bfloat16),
            pltpu.VMEM((BC, N), jnp.bfloat16),
            pltpu.VMEM((BC, N), jnp.bfloat16),
            pltpu.SemaphoreType.DMA((2, nblk)),
        ],
        compiler_params=pltpu.CompilerParams(
            vmem_limit_bytes=60 * 1024 * 1024,
        ),
    )(upper_lp, lower_lp, xt, wt, wh)

    return jnp.transpose(out.reshape(B, C_out, N), (0, 2, 1))


# speedup vs baseline: 1.4036x; 1.0217x over previous
import functools

import jax
import jax.numpy as jnp
from jax.experimental import pallas as pl
from jax.experimental.pallas import tpu as pltpu

_BR = 512
_NS = 2

_NT = (((1,), (1,)), ((), ()))


def _airtnn_body(u_ref, l_ref, xt_ref, wt_ref, wh_ref, out_ref,
                 stage_ref, cache_ref, ys_ref, ya_ref, yb_ref, sem_ref,
                 *, nblk, taps):
    mrefs = (u_ref, l_ref)

    def copy(m, i):
        return pltpu.make_async_copy(
            mrefs[m].at[pl.ds(i * _BR, _BR), :],
            stage_ref.at[(m * nblk + i) % _NS],
            sem_ref.at[m, i],
        )

    def land(m, i):
        copy(m, i).wait()
        s = m * nblk + i
        cache_ref[pl.ds(i * _BR, _BR), :] = stage_ref[s % _NS].astype(
            jnp.bfloat16)
        if s + _NS < 2 * nblk:
            copy(*divmod(s + _NS, nblk)).start()

    def tap(m, t, rdb, i, write_to=None, br=_BR):
        cols = pl.ds(i * br, br)
        yf = jax.lax.dot_general(rdb[:], cache_ref[pl.ds(i * br, br), :], _NT,
                                 preferred_element_type=jnp.float32)
        if write_to is not None:
            write_to[:, cols] = yf.astype(jnp.bfloat16)
        contrib = jax.lax.dot(wt_ref[m, t], yf,
                              preferred_element_type=jnp.float32)
        if m == 0 and t == 0:
            out_ref[:, cols] = contrib + jax.lax.dot(
                wh_ref[:], xt_ref[:, cols],
                preferred_element_type=jnp.float32)
        else:
            out_ref[:, cols] = out_ref[:, cols] + contrib

    for s in range(_NS):
        copy(*divmod(s, nblk)).start()

    ys_ref[:] = xt_ref[:].astype(jnp.bfloat16)

    for i in range(nblk):
        land(0, i)
        tap(0, 0, ys_ref, i, write_to=ya_ref)

    rdb, wrb = ya_ref, yb_ref
    for t in range(1, taps - 1):
        for i in range(nblk // 2):
            tap(0, t, rdb, i, write_to=wrb, br=2 * _BR)
        rdb, wrb = wrb, rdb

    for i in range(nblk):
        tap(0, taps - 1, rdb, i)
        land(1, i)
        tap(1, 0, ys_ref, i, write_to=ya_ref)

    rdb, wrb = ya_ref, yb_ref
    for t in range(1, taps):
        for i in range(nblk // 2):
            tap(1, t, rdb, i, write_to=wrb if t < taps - 1 else None,
                br=2 * _BR)
        rdb, wrb = wrb, rdb


def kernel(x, lower_lp, upper_lp, W_up, W_low, W_h):
    B, N, C_in = x.shape
    T, C_out, _ = W_up.shape
    BC = B * C_in
    BCO = B * C_out
    nblk = N // _BR

    xt = jnp.transpose(x, (0, 2, 1)).reshape(BC, N)

    eye = jnp.eye(B, dtype=jnp.float32)
    wt = jnp.stack([
        jnp.stack([jnp.kron(eye, W_up[t]) for t in range(T)]),
        jnp.stack([jnp.kron(eye, W_low[t]) for t in range(T)]),
    ])
    wh = jnp.kron(eye, W_h)

    out = pl.pallas_call(
        functools.partial(_airtnn_body, nblk=nblk, taps=T),
        in_specs=[
            pl.BlockSpec(memory_space=pl.ANY),
            pl.BlockSpec(memory_space=pl.ANY),
            pl.BlockSpec((BC, N), lambda: (0, 0)),
            pl.BlockSpec((2, T, BCO, BC), lambda: (0, 0, 0, 0)),
            pl.BlockSpec((BCO, BC), lambda: (0, 0)),
        ],
        out_specs=pl.BlockSpec((BCO, N), lambda: (0, 0)),
        out_shape=jax.ShapeDtypeStruct((BCO, N), jnp.float32),
        scratch_shapes=[
            pltpu.VMEM((_NS, _BR, N), jnp.float32),
            pltpu.VMEM((nblk * _BR, N), jnp.bfloat16),
            pltpu.VMEM((BC, N), jnp.bfloat16),
            pltpu.VMEM((BC, N), jnp.bfloat16),
            pltpu.VMEM((BC, N), jnp.bfloat16),
            pltpu.SemaphoreType.DMA((2, nblk)),
        ],
        compiler_params=pltpu.CompilerParams(
            vmem_limit_bytes=60 * 1024 * 1024,
        ),
    )(upper_lp, lower_lp, xt, wt, wh)

    return jnp.transpose(out.reshape(B, C_out, N), (0, 2, 1))
